# Initial kernel scaffold; baseline (speedup 1.0000x reference)
import jax
import jax.numpy as jnp
from jax.experimental import pallas as pl


def kernel(pos_w, pos_v, neg_w, neg_v, w_emb, v_emb):
    raise NotImplementedError("write your pallas kernel here")


# speedup vs baseline: 1.2829x; 1.2829x over previous
import functools

import jax
import jax.numpy as jnp
from jax import lax
from jax.experimental import pallas as pl
from jax.experimental.pallas import tpu as pltpu
from jax.experimental.pallas import tpu_sc as plsc

EMB_DIM = 128
B_POS = 16384
B_NEG = 81920
B_TOT = B_POS + B_NEG

NC = 2
NS = 16
NW = NC * NS
PER_W = B_TOT // NW
CHUNK = 128
NCHUNK = PER_W // CHUNK
GROUPS = CHUNK // 16

_mesh = plsc.VectorSubcoreMesh(core_axis_name="c", subcore_axis_name="s")


@functools.partial(
    pl.kernel,
    mesh=_mesh,
    out_type=jax.ShapeDtypeStruct((B_TOT,), jnp.float32),
    scratch_types=[
        pltpu.VMEM((CHUNK,), jnp.int32),
        pltpu.VMEM((CHUNK,), jnp.int32),
        pltpu.VMEM((CHUNK, EMB_DIM), jnp.float32),
        pltpu.VMEM((CHUNK, EMB_DIM), jnp.float32),
        pltpu.VMEM((256,), jnp.float32),
        pltpu.VMEM((CHUNK,), jnp.float32),
        pltpu.SemaphoreType.DMA,
        pltpu.SemaphoreType.DMA,
    ],
    compiler_params=pltpu.CompilerParams(needs_layout_passes=False),
)
def _sc_scores(idx_w_hbm, idx_v_hbm, w_hbm, v_hbm, out_hbm,
               idxw_v, idxv_v, wrows, vrows, pbuf, sc_v, sem_w, sem_v):
    wid = lax.axis_index("s") * NC + lax.axis_index("c")
    base = wid * PER_W
    lane = lax.broadcasted_iota(jnp.int32, (16,), 0)

    def chunk_body(c, carry):
        off = base + c * CHUNK
        pltpu.sync_copy(idx_w_hbm.at[pl.ds(off, CHUNK)], idxw_v)
        pltpu.sync_copy(idx_v_hbm.at[pl.ds(off, CHUNK)], idxv_v)
        cw = pltpu.async_copy(w_hbm.at[idxw_v], wrows, sem_w)
        cv = pltpu.async_copy(v_hbm.at[idxv_v], vrows, sem_v)
        cw.wait()
        cv.wait()

        def group_body(g, carry2):
            for i in range(16):
                p = g * 16 + i
                part = (wrows[p, pl.ds(0, 16)] * vrows[p, pl.ds(0, 16)])
                for j in range(1, 8):
                    part = part + (wrows[p, pl.ds(j * 16, 16)]
                                   * vrows[p, pl.ds(j * 16, 16)])
                pbuf[pl.ds(i * 16, 16)] = part
            acc = plsc.load_gather(pbuf, [lane * 16])
            for j in range(1, 16):
                acc = acc + plsc.load_gather(pbuf, [lane * 16 + j])
            sc_v[pl.ds(g * 16, 16)] = acc
            return carry2

        lax.fori_loop(0, GROUPS, group_body, 0)
        pltpu.sync_copy(sc_v, out_hbm.at[pl.ds(off, CHUNK)])
        return carry

    lax.fori_loop(0, NCHUNK, chunk_body, 0)


_ROWS = B_TOT // 128
_POS_ROWS = B_POS // 128


def _loss_body(s_ref, o_ref):
    s = s_ref[...]
    r = lax.broadcasted_iota(jnp.int32, (_ROWS, 128), 0)
    t = jnp.where(r < _POS_ROWS, -s, s)
    ls = jnp.minimum(t, 0.0) - jnp.log1p(jnp.exp(-jnp.abs(t)))
    o_ref[0, 0] = -jnp.sum(ls)


def kernel(pos_w, pos_v, neg_w, neg_v, w_emb, v_emb):
    idx_w = jnp.concatenate([pos_w, neg_w]).astype(jnp.int32)
    idx_v = jnp.concatenate([pos_v, neg_v]).astype(jnp.int32)
    scores = _sc_scores(idx_w, idx_v, w_emb, v_emb)
    loss = pl.pallas_call(
        _loss_body,
        out_shape=jax.ShapeDtypeStruct((1, 1), jnp.float32),
        out_specs=pl.BlockSpec(memory_space=pltpu.SMEM),
    )(scores.reshape(_ROWS, 128))
    return loss[0, 0]


# speedup vs baseline: 2.2726x; 1.7714x over previous
import functools

import jax
import jax.numpy as jnp
from jax import lax
from jax.experimental import pallas as pl
from jax.experimental.pallas import tpu as pltpu
from jax.experimental.pallas import tpu_sc as plsc

EMB_DIM = 128
B_POS = 16384
B_NEG = 81920
B_TOT = B_POS + B_NEG

NC = 2
NS = 16
NW = NC * NS
PER_W = B_TOT // NW
CHUNK = 128
NCHUNK = PER_W // CHUNK
GROUPS = CHUNK // 16

_mesh = plsc.VectorSubcoreMesh(core_axis_name="c", subcore_axis_name="s")


@functools.partial(
    pl.kernel,
    mesh=_mesh,
    out_type=jax.ShapeDtypeStruct((B_TOT,), jnp.float32),
    scratch_types=[
        pltpu.VMEM((PER_W,), jnp.int32),
        pltpu.VMEM((PER_W,), jnp.int32),
        pltpu.VMEM((CHUNK, EMB_DIM), jnp.float32),
        pltpu.VMEM((CHUNK, EMB_DIM), jnp.float32),
        pltpu.VMEM((CHUNK, EMB_DIM), jnp.float32),
        pltpu.VMEM((CHUNK, EMB_DIM), jnp.float32),
        pltpu.VMEM((256,), jnp.float32),
        pltpu.VMEM((PER_W,), jnp.float32),
        pltpu.SemaphoreType.DMA,
        pltpu.SemaphoreType.DMA,
        pltpu.SemaphoreType.DMA,
        pltpu.SemaphoreType.DMA,
    ],
    compiler_params=pltpu.CompilerParams(needs_layout_passes=False),
)
def _sc_scores(idx_w_hbm, idx_v_hbm, w_hbm, v_hbm, out_hbm,
               idxw_v, idxv_v, wrows0, vrows0, wrows1, vrows1, pbuf, sc_v,
               sem_w0, sem_v0, sem_w1, sem_v1):
    wid = lax.axis_index("s") * NC + lax.axis_index("c")
    base = wid * PER_W
    lane = lax.broadcasted_iota(jnp.int32, (16,), 0)

    pltpu.sync_copy(idx_w_hbm.at[pl.ds(base, PER_W)], idxw_v)
    pltpu.sync_copy(idx_v_hbm.at[pl.ds(base, PER_W)], idxv_v)

    bufs = ((wrows0, vrows0, sem_w0, sem_v0),
            (wrows1, vrows1, sem_w1, sem_v1))

    def gather_start(c, slot):
        w_r, v_r, s_w, s_v = bufs[slot]
        cw = pltpu.async_copy(
            w_hbm.at[idxw_v.at[pl.ds(c * CHUNK, CHUNK)]], w_r, s_w)
        cv = pltpu.async_copy(
            v_hbm.at[idxv_v.at[pl.ds(c * CHUNK, CHUNK)]], v_r, s_v)
        return cw, cv

    def gather_wait(c, slot):
        w_r, v_r, s_w, s_v = bufs[slot]
        pltpu.make_async_copy(
            w_hbm.at[idxw_v.at[pl.ds(c * CHUNK, CHUNK)]], w_r, s_w).wait()
        pltpu.make_async_copy(
            v_hbm.at[idxv_v.at[pl.ds(c * CHUNK, CHUNK)]], v_r, s_v).wait()

    def compute(c, slot):
        wrows, vrows, _, _ = bufs[slot]

        def group_body(g, carry2):
            for i in range(16):
                p = g * 16 + i
                pa = (wrows[p, pl.ds(0, 16)] * vrows[p, pl.ds(0, 16)])
                pb = (wrows[p, pl.ds(64, 16)] * vrows[p, pl.ds(64, 16)])
                for j in range(1, 4):
                    pa = pa + (wrows[p, pl.ds(j * 16, 16)]
                               * vrows[p, pl.ds(j * 16, 16)])
                    pb = pb + (wrows[p, pl.ds(64 + j * 16, 16)]
                               * vrows[p, pl.ds(64 + j * 16, 16)])
                pbuf[pl.ds(i * 16, 16)] = pa + pb
            acc0 = plsc.load_gather(pbuf, [lane * 16])
            acc1 = plsc.load_gather(pbuf, [lane * 16 + 1])
            for j in range(2, 16, 2):
                acc0 = acc0 + plsc.load_gather(pbuf, [lane * 16 + j])
                acc1 = acc1 + plsc.load_gather(pbuf, [lane * 16 + j + 1])
            sc_v[pl.ds(c * CHUNK + g * 16, 16)] = acc0 + acc1
            return carry2

        lax.fori_loop(0, GROUPS, group_body, 0)

    gather_start(0, 0)

    def pipe_body(cc, carry):
        c = cc * 2
        gather_start(c + 1, 1)
        gather_wait(c, 0)
        compute(c, 0)

        @pl.when(cc + 1 < NCHUNK // 2)
        def _():
            gather_start(c + 2, 0)

        gather_wait(c + 1, 1)
        compute(c + 1, 1)
        return carry

    lax.fori_loop(0, NCHUNK // 2, pipe_body, 0)
    pltpu.sync_copy(sc_v, out_hbm.at[pl.ds(base, PER_W)])


_ROWS = B_TOT // 128
_POS_ROWS = B_POS // 128


def _loss_body(s_ref, o_ref):
    s = s_ref[...]
    r = lax.broadcasted_iota(jnp.int32, (_ROWS, 128), 0)
    t = jnp.where(r < _POS_ROWS, -s, s)
    ls = jnp.minimum(t, 0.0) - jnp.log1p(jnp.exp(-jnp.abs(t)))
    o_ref[0, 0] = -jnp.sum(ls)


def kernel(pos_w, pos_v, neg_w, neg_v, w_emb, v_emb):
    idx_w = jnp.concatenate([pos_w, neg_w]).astype(jnp.int32)
    idx_v = jnp.concatenate([pos_v, neg_v]).astype(jnp.int32)
    scores = _sc_scores(idx_w, idx_v, w_emb, v_emb)
    loss = pl.pallas_call(
        _loss_body,
        out_shape=jax.ShapeDtypeStruct((1, 1), jnp.float32),
        out_specs=pl.BlockSpec(memory_space=pltpu.SMEM),
    )(scores.reshape(_ROWS, 128))
    return loss[0, 0]
